# baseline (speedup 1.0000x reference)
import functools

import jax
import jax.numpy as jnp
from jax import lax
from jax.experimental import pallas as pl
from jax.experimental.pallas import tpu as pltpu
from jax.experimental.pallas import tpu_sc as plsc

G = 100
V = 1000
H = 16
S = 128
BATCH = 16384

NW = 32
NB = 16

_mesh = plsc.VectorSubcoreMesh(core_axis_name="c", subcore_axis_name="s")


def _make_sc_gather(nrows):
    per_w = nrows // NW
    nstep = per_w // NB

    @functools.partial(
        pl.kernel,
        mesh=_mesh,
        compiler_params=pltpu.CompilerParams(use_tc_tiling_on_sc=False),
        out_type=jax.ShapeDtypeStruct((nrows * G, H), jnp.float32),
        scratch_types=[
            pltpu.VMEM((per_w, G), jnp.int32),
            pltpu.VMEM((NB * G, H), jnp.float32),
            pltpu.VMEM((NB * G, H), jnp.float32),
            pltpu.SemaphoreType.DMA,
            pltpu.SemaphoreType.DMA,
        ],
    )
    def _sc_gather(tab_hbm, idx_hbm, out_hbm, idx_v, rows_a, rows_b, sem_a, sem_b):
        wid = lax.axis_index("s") * 2 + lax.axis_index("c")
        b0 = wid * per_w

        pltpu.sync_copy(idx_hbm.at[pl.ds(b0, per_w)], idx_v)

        def fire(c, rows_v, sem):
            for j in range(NB):
                pltpu.async_copy(
                    tab_hbm.at[idx_v.at[c * NB + j]],
                    rows_v.at[pl.ds(j * G, G)],
                    sem)

        def drain_store(c, rows_v, sem):
            for j in range(NB):
                pltpu.make_async_copy(
                    tab_hbm.at[idx_v.at[c * NB + j]],
                    rows_v.at[pl.ds(j * G, G)],
                    sem).wait()
            pltpu.sync_copy(
                rows_v,
                out_hbm.at[pl.ds((b0 + c * NB) * G, NB * G)])

        fire(0, rows_a, sem_a)

        def step(c, carry):
            @pl.when(c % 2 == 0)
            def _():
                @pl.when(c + 1 < nstep)
                def _():
                    fire(c + 1, rows_b, sem_b)
                drain_store(c, rows_a, sem_a)

            @pl.when(c % 2 == 1)
            def _():
                @pl.when(c + 1 < nstep)
                def _():
                    fire(c + 1, rows_a, sem_a)
                drain_store(c, rows_b, sem_b)

            return carry

        lax.fori_loop(0, nstep, step, 0)

    return _sc_gather


BM = 512
NCHUNK = 8
CH = BATCH // NCHUNK


def _mm_body(a_ref, w_ref, bias_ref, o_ref):
    o_ref[...] = jnp.dot(a_ref[...].astype(jnp.bfloat16), w_ref[...],
                         preferred_element_type=jnp.float32) + bias_ref[...]


def _matmul(a, w, bias):
    rows = a.shape[0]
    return pl.pallas_call(
        _mm_body,
        grid=(rows // BM,),
        in_specs=[
            pl.BlockSpec((BM, G * H), lambda i: (i, 0)),
            pl.BlockSpec((G * H, S), lambda i: (0, 0)),
            pl.BlockSpec((1, S), lambda i: (0, 0)),
        ],
        out_specs=pl.BlockSpec((BM, S), lambda i: (i, 0)),
        out_shape=jax.ShapeDtypeStruct((rows, S), jnp.float32),
    )(a, w, bias)


_gather_chunk = _make_sc_gather(CH)


def kernel(x, tables, W, b):
    flat_tables = tables.reshape(G * V, H)
    gidx = x + (jnp.arange(G, dtype=jnp.int32) * V)[None, :]
    wb = W.astype(jnp.bfloat16)
    b2 = b.reshape(1, S)
    outs = []
    for ci in range(NCHUNK):
        a = _gather_chunk(flat_tables,
                          lax.slice(gidx, (ci * CH, 0), ((ci + 1) * CH, G)))
        outs.append(_matmul(a.reshape(CH, G * H), wb, b2))
    return jnp.concatenate(outs, axis=0)


# speedup vs baseline: 1.0366x; 1.0366x over previous
import functools

import jax
import jax.numpy as jnp
from jax import lax
from jax.experimental import pallas as pl
from jax.experimental.pallas import tpu as pltpu
from jax.experimental.pallas import tpu_sc as plsc

G = 100
V = 1000
H = 16
S = 128
BATCH = 16384

NW = 32
NB = 16

_mesh = plsc.VectorSubcoreMesh(core_axis_name="c", subcore_axis_name="s")


def _make_sc_gather(nrows):
    per_w = nrows // NW
    nstep = per_w // NB

    @functools.partial(
        pl.kernel,
        mesh=_mesh,
        compiler_params=pltpu.CompilerParams(use_tc_tiling_on_sc=False),
        out_type=jax.ShapeDtypeStruct((nrows * G, H), jnp.float32),
        scratch_types=[
            pltpu.VMEM((per_w, G), jnp.int32),
            pltpu.VMEM((NB * G, H), jnp.float32),
            pltpu.VMEM((NB * G, H), jnp.float32),
            pltpu.SemaphoreType.DMA,
            pltpu.SemaphoreType.DMA,
        ],
    )
    def _sc_gather(tab_hbm, idx_hbm, out_hbm, idx_v, rows_a, rows_b, sem_a, sem_b):
        wid = lax.axis_index("s") * 2 + lax.axis_index("c")
        b0 = wid * per_w

        pltpu.sync_copy(idx_hbm.at[pl.ds(b0, per_w)], idx_v)

        def fire(c, rows_v, sem):
            for j in range(NB):
                pltpu.async_copy(
                    tab_hbm.at[idx_v.at[c * NB + j]],
                    rows_v.at[pl.ds(j * G, G)],
                    sem)

        def drain_store(c, rows_v, sem):
            for j in range(NB):
                pltpu.make_async_copy(
                    tab_hbm.at[idx_v.at[c * NB + j]],
                    rows_v.at[pl.ds(j * G, G)],
                    sem).wait()
            pltpu.sync_copy(
                rows_v,
                out_hbm.at[pl.ds((b0 + c * NB) * G, NB * G)])

        fire(0, rows_a, sem_a)

        def step(c, carry):
            @pl.when(c % 2 == 0)
            def _():
                @pl.when(c + 1 < nstep)
                def _():
                    fire(c + 1, rows_b, sem_b)
                drain_store(c, rows_a, sem_a)

            @pl.when(c % 2 == 1)
            def _():
                @pl.when(c + 1 < nstep)
                def _():
                    fire(c + 1, rows_a, sem_a)
                drain_store(c, rows_b, sem_b)

            return carry

        lax.fori_loop(0, nstep, step, 0)

    return _sc_gather


BM = 512
NCHUNK = 2
CH = BATCH // NCHUNK


def _mm_body(a_ref, w_ref, bias_ref, o_ref):
    o_ref[...] = jnp.dot(a_ref[...].astype(jnp.bfloat16), w_ref[...],
                         preferred_element_type=jnp.float32) + bias_ref[...]


def _matmul(a, w, bias):
    rows = a.shape[0]
    return pl.pallas_call(
        _mm_body,
        grid=(rows // BM,),
        in_specs=[
            pl.BlockSpec((BM, G * H), lambda i: (i, 0)),
            pl.BlockSpec((G * H, S), lambda i: (0, 0)),
            pl.BlockSpec((1, S), lambda i: (0, 0)),
        ],
        out_specs=pl.BlockSpec((BM, S), lambda i: (i, 0)),
        out_shape=jax.ShapeDtypeStruct((rows, S), jnp.float32),
    )(a, w, bias)


_gather_chunk = _make_sc_gather(CH)


def kernel(x, tables, W, b):
    flat_tables = tables.reshape(G * V, H)
    gidx = x + (jnp.arange(G, dtype=jnp.int32) * V)[None, :]
    wb = W.astype(jnp.bfloat16)
    b2 = b.reshape(1, S)
    outs = []
    for ci in range(NCHUNK):
        a = _gather_chunk(flat_tables,
                          lax.slice(gidx, (ci * CH, 0), ((ci + 1) * CH, G)))
        outs.append(_matmul(a.reshape(CH, G * H), wb, b2))
    return jnp.concatenate(outs, axis=0)


# speedup vs baseline: 1.0381x; 1.0015x over previous
import functools

import jax
import jax.numpy as jnp
from jax import lax
from jax.experimental import pallas as pl
from jax.experimental.pallas import tpu as pltpu
from jax.experimental.pallas import tpu_sc as plsc

G = 100
V = 1000
H = 16
S = 128
BATCH = 16384

NW = 32
NB = 32

_mesh = plsc.VectorSubcoreMesh(core_axis_name="c", subcore_axis_name="s")


def _make_sc_gather(nrows):
    per_w = nrows // NW
    nstep = per_w // NB

    @functools.partial(
        pl.kernel,
        mesh=_mesh,
        compiler_params=pltpu.CompilerParams(use_tc_tiling_on_sc=False),
        out_type=jax.ShapeDtypeStruct((nrows * G, H), jnp.float32),
        scratch_types=[
            pltpu.VMEM((per_w, G), jnp.int32),
            pltpu.VMEM((NB * G, H), jnp.float32),
            pltpu.VMEM((NB * G, H), jnp.float32),
            pltpu.SemaphoreType.DMA,
            pltpu.SemaphoreType.DMA,
        ],
    )
    def _sc_gather(tab_hbm, idx_hbm, out_hbm, idx_v, rows_a, rows_b, sem_a, sem_b):
        wid = lax.axis_index("s") * 2 + lax.axis_index("c")
        b0 = wid * per_w

        pltpu.sync_copy(idx_hbm.at[pl.ds(b0, per_w)], idx_v)

        def fire(c, rows_v, sem):
            for j in range(NB):
                pltpu.async_copy(
                    tab_hbm.at[idx_v.at[c * NB + j]],
                    rows_v.at[pl.ds(j * G, G)],
                    sem)

        def drain_store(c, rows_v, sem):
            for j in range(NB):
                pltpu.make_async_copy(
                    tab_hbm.at[idx_v.at[c * NB + j]],
                    rows_v.at[pl.ds(j * G, G)],
                    sem).wait()
            pltpu.sync_copy(
                rows_v,
                out_hbm.at[pl.ds((b0 + c * NB) * G, NB * G)])

        fire(0, rows_a, sem_a)

        def step(c, carry):
            @pl.when(c % 2 == 0)
            def _():
                @pl.when(c + 1 < nstep)
                def _():
                    fire(c + 1, rows_b, sem_b)
                drain_store(c, rows_a, sem_a)

            @pl.when(c % 2 == 1)
            def _():
                @pl.when(c + 1 < nstep)
                def _():
                    fire(c + 1, rows_a, sem_a)
                drain_store(c, rows_b, sem_b)

            return carry

        lax.fori_loop(0, nstep, step, 0)

    return _sc_gather


BM = 512
NCHUNK = 2
CH = BATCH // NCHUNK


def _mm_body(a_ref, w_ref, bias_ref, o_ref):
    o_ref[...] = jnp.dot(a_ref[...].astype(jnp.bfloat16), w_ref[...],
                         preferred_element_type=jnp.float32) + bias_ref[...]


def _matmul(a, w, bias):
    rows = a.shape[0]
    return pl.pallas_call(
        _mm_body,
        grid=(rows // BM,),
        in_specs=[
            pl.BlockSpec((BM, G * H), lambda i: (i, 0)),
            pl.BlockSpec((G * H, S), lambda i: (0, 0)),
            pl.BlockSpec((1, S), lambda i: (0, 0)),
        ],
        out_specs=pl.BlockSpec((BM, S), lambda i: (i, 0)),
        out_shape=jax.ShapeDtypeStruct((rows, S), jnp.float32),
    )(a, w, bias)


_gather_chunk = _make_sc_gather(CH)


def kernel(x, tables, W, b):
    flat_tables = tables.reshape(G * V, H)
    gidx = x + (jnp.arange(G, dtype=jnp.int32) * V)[None, :]
    wb = W.astype(jnp.bfloat16)
    b2 = b.reshape(1, S)
    outs = []
    for ci in range(NCHUNK):
        a = _gather_chunk(flat_tables,
                          lax.slice(gidx, (ci * CH, 0), ((ci + 1) * CH, G)))
        outs.append(_matmul(a.reshape(CH, G * H), wb, b2))
    return jnp.concatenate(outs, axis=0)
